# baseline (speedup 1.0000x reference)
import functools

import jax
import jax.numpy as jnp
from jax import lax
from jax.experimental import pallas as pl
from jax.experimental.pallas import tpu as pltpu
from jax.experimental.pallas import tpu_sc as plsc

N_VOCAB = 100000
D_MODEL = 768
BOS = 0

NC = 2
NS = 16
NW = NC * NS

B = 4
T = 8193
CHUNK = 32
NBUF = 5
W_PER_B = NW // B
PER_W = 8192 // W_PER_B
CHUNKS_W = PER_W // CHUNK
T_PAD = T - 1 + CHUNK

_mesh = plsc.VectorSubcoreMesh(core_axis_name="c", subcore_axis_name="s")


@functools.partial(
    pl.kernel,
    out_type=jax.ShapeDtypeStruct((B, T, D_MODEL), jnp.float32),
    mesh=_mesh,
    scratch_types=[
        pltpu.VMEM((CHUNKS_W + 1, CHUNK), jnp.int32),
        [pltpu.VMEM((CHUNK, D_MODEL), jnp.float32) for _ in range(NBUF)],
        [pltpu.SemaphoreType.DMA for _ in range(NBUF)],
        [pltpu.SemaphoreType.DMA for _ in range(NBUF)],
    ],
)
def _gather_kernel(ids_hbm, table_hbm, out_hbm, idx_v, bufs, gsems, wsems):
    wid = lax.axis_index("s") * NC + lax.axis_index("c")
    b = wid // W_PER_B
    lane = wid % W_PER_B

    pltpu.sync_copy(ids_hbm.at[b, pl.ds(lane * CHUNKS_W, CHUNKS_W)],
                    idx_v.at[pl.ds(0, CHUNKS_W)])

    def gather(j):
        return pltpu.async_copy(table_hbm.at[idx_v.at[j]], bufs[j % NBUF],
                                gsems[j % NBUF])

    def write(j):
        return pltpu.async_copy(
            bufs[j % NBUF],
            out_hbm.at[b, pl.ds(lane * PER_W + j * CHUNK, CHUNK)],
            wsems[j % NBUF])

    gathers = [None] * CHUNKS_W
    writes = [None] * CHUNKS_W
    for j in range(3):
        gathers[j] = gather(j)
    for j in range(CHUNKS_W):
        if j + 3 < CHUNKS_W:
            if j >= 2:
                writes[j - 2].wait()
            gathers[j + 3] = gather(j + 3)
        gathers[j].wait()
        writes[j] = write(j)
    for j in range(CHUNKS_W - 5, CHUNKS_W):
        writes[j].wait()

    @pl.when(lane == W_PER_B - 1)
    def _tail():
        pltpu.sync_copy(ids_hbm.at[b, pl.ds(T_PAD // CHUNK - 1, 1)],
                        idx_v.at[pl.ds(CHUNKS_W, 1)])
        pltpu.async_copy(table_hbm.at[idx_v.at[CHUNKS_W]], bufs[0],
                         gsems[0]).wait()
        pltpu.sync_copy(bufs[0].at[pl.ds(0, 1)],
                        out_hbm.at[b, pl.ds(W_PER_B * PER_W, 1)])


def kernel(input_ids, embedding):
    ids = jnp.pad(input_ids.astype(jnp.int32), ((0, 0), (1, 0)),
                  constant_values=BOS)
    ids = jnp.pad(ids, ((0, 0), (0, T_PAD - T)), constant_values=0)
    return _gather_kernel(ids.reshape(B, T_PAD // CHUNK, CHUNK), embedding)
